# baseline (speedup 1.0000x reference)
import functools

import jax
import jax.numpy as jnp
import numpy as np
from jax import lax
from jax.experimental import pallas as pl
from jax.experimental.pallas import tpu as pltpu
from jax.experimental.pallas import tpu_sc as plsc

NUM_CLASSES = 80
NUM_ANCHORS = 20000
BATCH = 8
MAX_TRUE = 100
POS_THRESH = 0.5
NEG_THRESH = 0.4
ALPHA = 0.25
GAMMA = 2.0
EPS = 1e-7

AB = 4000
NBLK = NUM_ANCHORS // AB

TOTAL = BATCH * NUM_ANCHORS
NW = 32
CHUNK = TOTAL // NW
NV = CHUNK // 16
CHUNK_PAD = (NV + 1) * 16
CAPW = 5120
GB = 128

_ATAN_C = (1.00000000e+00, -3.33333331e-01, 1.99999846e-01, -1.42853316e-01,
           1.11062643e-01, -9.05436302e-02, 7.51323369e-02, -6.06349744e-02,
           4.42830421e-02, -2.66563540e-02, 1.18503795e-02, -3.35367563e-03,
           4.45197908e-04)


def _atan_nonneg(x):
    big = x > 1.0
    t = jnp.where(big, 1.0 / jnp.maximum(x, 1.0), x)
    t2 = t * t
    acc = jnp.full_like(t, _ATAN_C[-1])
    for c in _ATAN_C[-2::-1]:
        acc = acc * t2 + c
    a = t * acc
    return jnp.where(big, (np.pi / 2.0) - a, a)


def _k1_block(ancT_ref, bt_ref, btT_ref, yt_ref, conf_ref, bpT_ref,
              acc_ref, flag_ref, ca_ref):
    b = pl.program_id(0)
    j = pl.program_id(1)
    anc = ancT_ref[...]
    bt = bt_ref[0]
    btT = btT_ref[0]
    yt = yt_ref[0]
    conf = conf_ref[0, 0]
    bp = bpT_ref[0]

    ax1 = anc[0, 0]; ay1 = anc[1, 0]; ax2 = anc[2, 0]; ay2 = anc[3, 0]
    bx1 = bt[:, 0:1]; by1 = bt[:, 1:2]; bx2 = bt[:, 2:3]; by2 = bt[:, 3:4]

    ix1 = jnp.maximum(ax1, bx1)
    iy1 = jnp.maximum(ay1, by1)
    ix2 = jnp.minimum(ax2, bx2)
    iy2 = jnp.minimum(ay2, by2)
    inter = jnp.clip(ix2 - ix1, 0.0) * jnp.clip(iy2 - iy1, 0.0)
    area_a = jnp.clip(ax2 - ax1, 0.0) * jnp.clip(ay2 - ay1, 0.0)
    area_b = jnp.clip(bx2 - bx1, 0.0) * jnp.clip(by2 - by1, 0.0)
    iou = inter / (area_a + area_b - inter + EPS)

    max_iou = jnp.max(iou, axis=0, keepdims=True)
    pos = max_iou >= POS_THRESH
    neg = max_iou < NEG_THRESH
    pw = pos.astype(jnp.float32)
    tw = (pos | neg).astype(jnp.float32)
    possel = (iou == max_iou).astype(jnp.float32) * pw

    p = jnp.clip(conf, EPS, 1.0 - EPS)
    bce = -(pw * jnp.log(p) + (1.0 - pw) * jnp.log(1.0 - p))
    s_sum = jnp.sum(bce * tw)

    kvec = jax.lax.broadcasted_iota(jnp.int32, (NUM_CLASSES, 1), 0)
    cls_col = jnp.dot(yt, kvec.astype(jnp.float32),
                      preferred_element_type=jnp.float32)
    c_a = jax.lax.dot_general(cls_col, possel, (((0,), (0,)), ((), ())),
                              preferred_element_type=jnp.float32)

    basn = jnp.dot(btT, possel, preferred_element_type=jnp.float32)
    x1t = basn[0:1, :]; y1t = basn[1:2, :]; x2t = basn[2:3, :]; y2t = basn[3:4, :]
    x1p = bp[0, 0]; y1p = bp[1, 0]; x2p = bp[2, 0]; y2p = bp[3, 0]
    wt = jnp.clip(x2t - x1t, 0.0); ht = jnp.clip(y2t - y1t, 0.0)
    wp = jnp.clip(x2p - x1p, 0.0); hp = jnp.clip(y2p - y1p, 0.0)
    inter2 = jnp.clip(jnp.minimum(x2t, x2p) - jnp.maximum(x1t, x1p), 0.0) * \
             jnp.clip(jnp.minimum(y2t, y2p) - jnp.maximum(y1t, y1p), 0.0)
    union = wt * ht + wp * hp - inter2
    iou2 = inter2 / (union + EPS)
    cw = jnp.maximum(x2t, x2p) - jnp.minimum(x1t, x1p)
    ch = jnp.maximum(y2t, y2p) - jnp.minimum(y1t, y1p)
    c2 = cw * cw + ch * ch + EPS
    rho2 = ((x1t + x2t - x1p - x2p) ** 2 + (y1t + y2t - y1p - y2p) ** 2) / 4.0
    v = (4.0 / (np.pi ** 2)) * (_atan_nonneg(wt / (ht + EPS)) -
                                _atan_nonneg(wp / (hp + EPS))) ** 2
    alpha_t = v / (1.0 - iou2 + v + EPS)
    cl = 1.0 - (iou2 - rho2 / c2 - alpha_t * v)
    b_sum = jnp.sum(cl * pw)

    cnt = jnp.sum(pw)

    rows = jnp.concatenate([
        jnp.full((1, 128), s_sum, jnp.float32),
        jnp.full((1, 128), 0.0, jnp.float32),
        jnp.full((1, 128), b_sum, jnp.float32),
        jnp.full((1, 128), cnt, jnp.float32),
        jnp.zeros((4, 128), jnp.float32),
    ], axis=0)

    @pl.when((b == 0) & (j == 0))
    def _init():
        acc_ref[...] = jnp.zeros_like(acc_ref)

    acc_ref[b] = acc_ref[b] + rows
    flag_ref[0, 0] = pw
    ca_ref[0, 0] = c_a


def _run_k1(y_true, bbox_true, conf_pred, bbox_pred, anchors):
    ancT = jnp.transpose(anchors, (1, 0)).reshape(4, NBLK, 1, AB)
    btT = jnp.transpose(bbox_true, (0, 2, 1))
    conf3 = conf_pred.reshape(BATCH, NBLK, 1, AB)
    bpT = jnp.transpose(bbox_pred, (0, 2, 1)).reshape(BATCH, 4, NBLK, 1, AB)

    return pl.pallas_call(
        _k1_block,
        grid=(BATCH, NBLK),
        in_specs=[
            pl.BlockSpec((4, 1, 1, AB), lambda b, j: (0, j, 0, 0)),
            pl.BlockSpec((1, MAX_TRUE, 4), lambda b, j: (b, 0, 0)),
            pl.BlockSpec((1, 4, MAX_TRUE), lambda b, j: (b, 0, 0)),
            pl.BlockSpec((1, MAX_TRUE, NUM_CLASSES), lambda b, j: (b, 0, 0)),
            pl.BlockSpec((1, 1, 1, AB), lambda b, j: (b, j, 0, 0)),
            pl.BlockSpec((1, 4, 1, 1, AB), lambda b, j: (b, 0, j, 0, 0)),
        ],
        out_specs=[
            pl.BlockSpec((BATCH, 8, 128), lambda b, j: (0, 0, 0)),
            pl.BlockSpec((1, 1, 1, AB), lambda b, j: (b, j, 0, 0)),
            pl.BlockSpec((1, 1, 1, AB), lambda b, j: (b, j, 0, 0)),
        ],
        out_shape=[
            jax.ShapeDtypeStruct((BATCH, 8, 128), jnp.float32),
            jax.ShapeDtypeStruct((BATCH, NBLK, 1, AB), jnp.float32),
            jax.ShapeDtypeStruct((BATCH, NBLK, 1, AB), jnp.float32),
        ],
    )(ancT, bbox_true, btT, y_true, conf3, bpT)


def _sc_body(flags_hbm, ca_hbm, logit_hbm, rows_out, ca_out, cnt_out,
             flg_v, cav_v, idx_v, cac_v, idxw_v, rows_v, cnt_v, sem):
    wid = lax.axis_index("s") * 2 + lax.axis_index("c")
    base = wid * CHUNK

    zeros16 = jnp.zeros((16,), jnp.float32)
    flg_v[pl.ds(NV * 16, 16)] = zeros16
    pltpu.sync_copy(flags_hbm.at[pl.ds(base, CHUNK)], flg_v.at[pl.ds(0, CHUNK)])
    pltpu.sync_copy(ca_hbm.at[pl.ds(base, CHUNK)], cav_v.at[pl.ds(0, CHUNK)])

    def zbody(i, _):
        idx_v[pl.ds(i * 16, 16)] = jnp.zeros((16,), jnp.int32)
        return 0
    lax.fori_loop(0, (CAPW + 16) // 16, zbody, 0)

    lane = lax.broadcasted_iota(jnp.int32, (16,), 0)

    def cbody(i, off):
        v = flg_v[pl.ds(i * 16, 16)]
        cav = cav_v[pl.ds(i * 16, 16)]
        m = v > 0.5
        idxs = base + i * 16 + lane
        plsc.store_compressed(idx_v.at[pl.ds(off, 16)], idxs, mask=m)
        plsc.store_compressed(cac_v.at[pl.ds(off, 16)], cav, mask=m)
        return off + plsc.all_reduce_population_count(m)[0]
    p_w = lax.fori_loop(0, NV + 1, cbody, 0)

    nch = (p_w + (GB - 1)) // GB

    def gbody(k, _):
        def wbody(jj, _2):
            idxw_v[pl.ds(jj * 16, 16)] = idx_v[pl.ds(k * GB + jj * 16, 16)]
            return 0
        lax.fori_loop(0, GB // 16, wbody, 0)
        pltpu.async_copy(logit_hbm.at[idxw_v], rows_v, sem).wait()
        pltpu.sync_copy(rows_v, rows_out.at[wid, pl.ds(k * GB, GB)])
        pltpu.sync_copy(cac_v.at[pl.ds(k * GB, GB)],
                        ca_out.at[wid, pl.ds(k * GB, GB)])
        return 0
    lax.fori_loop(0, nch, gbody, 0)

    cnt_v[...] = jnp.full((16,), p_w, jnp.int32)
    pltpu.sync_copy(cnt_v, cnt_out.at[wid])


def _run_sc(flags_flat, ca_flat, logit_flat):
    mesh = plsc.VectorSubcoreMesh(core_axis_name="c", subcore_axis_name="s")
    kern = functools.partial(
        pl.kernel, mesh=mesh,
        compiler_params=pltpu.CompilerParams(
            use_tc_tiling_on_sc=False, needs_layout_passes=False),
        out_type=[
            jax.ShapeDtypeStruct((NW, CAPW, NUM_CLASSES), jnp.float32),
            jax.ShapeDtypeStruct((NW, CAPW), jnp.float32),
            jax.ShapeDtypeStruct((NW, 16), jnp.int32),
        ],
        scratch_types=[
            pltpu.VMEM((CHUNK_PAD,), jnp.float32),
            pltpu.VMEM((CHUNK_PAD,), jnp.float32),
            pltpu.VMEM((CAPW + 16,), jnp.int32),
            pltpu.VMEM((CAPW + 16,), jnp.float32),
            pltpu.VMEM((GB,), jnp.int32),
            pltpu.VMEM((GB, NUM_CLASSES), jnp.float32),
            pltpu.VMEM((16,), jnp.int32),
            pltpu.SemaphoreType.DMA,
        ],
    )(_sc_body)
    return kern(flags_flat, ca_flat, logit_flat)


def _k2_block(cnt_ref, acc_ref, rows_hbm, ca_hbm, out_ref,
              rows_v, cav_v, sem1, sem2):
    w = pl.program_id(0)
    cnt_w = cnt_ref[0, 0, 0]

    lane_k = jax.lax.broadcasted_iota(jnp.int32, (GB, NUM_CLASSES), 1)
    sub_i = jax.lax.broadcasted_iota(jnp.int32, (GB, 1), 0)

    def chunk_body(k, csum):
        cp1 = pltpu.make_async_copy(rows_hbm.at[w, pl.ds(k * GB, GB)],
                                    rows_v, sem1)
        cp2 = pltpu.make_async_copy(ca_hbm.at[w, pl.ds(k * GB, GB)],
                                    cav_v, sem2)
        cp1.start(); cp2.start()
        cp1.wait(); cp2.wait()
        q = jnp.clip(rows_v[...], EPS, 1.0 - EPS)
        r = 1.0 - q
        neg_term = -(1.0 - ALPHA) * q * q * jnp.log(r)
        pos_term = -ALPHA * r * r * jnp.log(q)
        ca = cav_v[...]
        onehot = (lane_k == ca.astype(jnp.int32)).astype(jnp.float32)
        row_sum = jnp.sum(neg_term, axis=1, keepdims=True) + \
                  jnp.sum((pos_term - neg_term) * onehot, axis=1, keepdims=True)
        mask = (k * GB + sub_i < cnt_w).astype(jnp.float32)
        return csum + jnp.sum(row_sum * mask)

    nch = (cnt_w + (GB - 1)) // GB
    csum = lax.fori_loop(0, nch, chunk_body, 0.0)

    @pl.when(w == 0)
    def _init():
        out_ref[...] = jnp.zeros_like(out_ref)

    out_ref[...] = out_ref[...] + jnp.full((1, 128), csum, jnp.float32)

    @pl.when(w == NW - 1)
    def _finalize():
        allp = acc_ref[...]
        cnts = allp[:, 3, 0:1]
        avg = jnp.sum(jnp.maximum(cnts, 1.0))
        sv = jnp.sum(allp[:, 0, 0:1]) / avg
        bv = jnp.sum(allp[:, 2, 0:1]) / avg
        cv = out_ref[0, 0] / avg
        lane = jax.lax.broadcasted_iota(jnp.int32, (1, 128), 1)
        row = jnp.where(lane == 0, sv,
                        jnp.where(lane == 1, cv,
                                  jnp.where(lane == 2, bv, 0.0)))
        row = jnp.where(jnp.isnan(row) | jnp.isinf(row), 0.0, row)
        out_ref[...] = row


def _run_k2(cnt, acc, rows, ca):
    cnt3 = cnt.reshape(NW, 1, 16)
    ca3 = ca.reshape(NW, CAPW, 1)
    return pl.pallas_call(
        _k2_block,
        grid=(NW,),
        in_specs=[
            pl.BlockSpec((1, 1, 16), lambda w: (w, 0, 0)),
            pl.BlockSpec((BATCH, 8, 128), lambda w: (0, 0, 0)),
            pl.BlockSpec(memory_space=pl.ANY),
            pl.BlockSpec(memory_space=pl.ANY),
        ],
        out_specs=pl.BlockSpec((1, 128), lambda w: (0, 0)),
        out_shape=jax.ShapeDtypeStruct((1, 128), jnp.float32),
        scratch_shapes=[
            pltpu.VMEM((GB, NUM_CLASSES), jnp.float32),
            pltpu.VMEM((GB, 1), jnp.float32),
            pltpu.SemaphoreType.DMA,
            pltpu.SemaphoreType.DMA,
        ],
    )(cnt3, acc, rows, ca3)


@jax.jit
def kernel(y_true, bbox_true, conf_pred, logit_pred, bbox_pred, anchors):
    acc, flags, ca = _run_k1(y_true, bbox_true, conf_pred, bbox_pred, anchors)
    flags_flat = flags.reshape(TOTAL)
    ca_flat = ca.reshape(TOTAL)
    logit_flat = logit_pred.reshape(TOTAL, NUM_CLASSES)
    rows, cac, cnt = _run_sc(flags_flat, ca_flat, logit_flat)
    out = _run_k2(cnt, acc, rows, cac)
    return out[0, :3]


# speedup vs baseline: 2.9721x; 2.9721x over previous
import jax
import jax.numpy as jnp
import numpy as np
from jax.experimental import pallas as pl

NUM_CLASSES = 80
NUM_ANCHORS = 20000
BATCH = 8
MAX_TRUE = 100
POS_THRESH = 0.5
NEG_THRESH = 0.4
ALPHA = 0.25
GAMMA = 2.0
EPS = 1e-7

AB = 4000
NBLK = NUM_ANCHORS // AB

_ATAN_C = (1.00000000e+00, -3.33333331e-01, 1.99999846e-01, -1.42853316e-01,
           1.11062643e-01, -9.05436302e-02, 7.51323369e-02, -6.06349744e-02,
           4.42830421e-02, -2.66563540e-02, 1.18503795e-02, -3.35367563e-03,
           4.45197908e-04)


def _atan_nonneg(x):
    big = x > 1.0
    t = jnp.where(big, 1.0 / jnp.maximum(x, 1.0), x)
    t2 = t * t
    acc = jnp.full_like(t, _ATAN_C[-1])
    for c in _ATAN_C[-2::-1]:
        acc = acc * t2 + c
    a = t * acc
    return jnp.where(big, (np.pi / 2.0) - a, a)


def _loss_block(ancT_ref, bt_ref, btT_ref, yt_ref, conf_ref, logit_ref,
                bpT_ref, out_ref):
    j = pl.program_id(1)
    anc = ancT_ref[...]
    bt = bt_ref[0]
    btT = btT_ref[0]
    yt = yt_ref[0]
    conf = conf_ref[0, 0]
    q_raw = logit_ref[0]
    bp = bpT_ref[0]

    ax1 = anc[0, 0]; ay1 = anc[1, 0]; ax2 = anc[2, 0]; ay2 = anc[3, 0]
    bx1 = bt[:, 0:1]; by1 = bt[:, 1:2]; bx2 = bt[:, 2:3]; by2 = bt[:, 3:4]

    ix1 = jnp.maximum(ax1, bx1)
    iy1 = jnp.maximum(ay1, by1)
    ix2 = jnp.minimum(ax2, bx2)
    iy2 = jnp.minimum(ay2, by2)
    inter = jnp.clip(ix2 - ix1, 0.0) * jnp.clip(iy2 - iy1, 0.0)
    area_a = jnp.clip(ax2 - ax1, 0.0) * jnp.clip(ay2 - ay1, 0.0)
    area_b = jnp.clip(bx2 - bx1, 0.0) * jnp.clip(by2 - by1, 0.0)
    iou = inter / (area_a + area_b - inter + EPS)

    valid = jnp.any(bt > 0, axis=1, keepdims=True)
    iou = jnp.where(valid, iou, -1.0)
    max_iou = jnp.max(iou, axis=0, keepdims=True)

    pos = max_iou >= POS_THRESH
    neg = max_iou < NEG_THRESH
    pw = pos.astype(jnp.float32)
    tw = (pos | neg).astype(jnp.float32)

    possel = (iou == max_iou).astype(jnp.float32) * pw

    p = jnp.clip(conf, EPS, 1.0 - EPS)
    bce = -(pw * jnp.log(p) + (1.0 - pw) * jnp.log(1.0 - p))
    s_sum = jnp.sum(bce * tw)

    q = jnp.clip(q_raw, EPS, 1.0 - EPS)
    r = 1.0 - q
    neg_term = -(1.0 - ALPHA) * q * q * jnp.log(r)
    pos_term = -ALPHA * r * r * jnp.log(q)
    h = pos_term - neg_term
    t1 = jnp.sum(jnp.dot(pw, neg_term, preferred_element_type=jnp.float32))
    g = jnp.dot(possel, h, preferred_element_type=jnp.float32)
    t2 = jnp.sum(g * yt)
    c_sum = t1 + t2

    basn = jnp.dot(btT, possel, preferred_element_type=jnp.float32)
    x1t = basn[0:1, :]; y1t = basn[1:2, :]; x2t = basn[2:3, :]; y2t = basn[3:4, :]
    x1p = bp[0, 0]; y1p = bp[1, 0]; x2p = bp[2, 0]; y2p = bp[3, 0]
    wt = jnp.clip(x2t - x1t, 0.0); ht = jnp.clip(y2t - y1t, 0.0)
    wp = jnp.clip(x2p - x1p, 0.0); hp = jnp.clip(y2p - y1p, 0.0)
    inter2 = jnp.clip(jnp.minimum(x2t, x2p) - jnp.maximum(x1t, x1p), 0.0) * \
             jnp.clip(jnp.minimum(y2t, y2p) - jnp.maximum(y1t, y1p), 0.0)
    union = wt * ht + wp * hp - inter2
    iou2 = inter2 / (union + EPS)
    cw = jnp.maximum(x2t, x2p) - jnp.minimum(x1t, x1p)
    ch = jnp.maximum(y2t, y2p) - jnp.minimum(y1t, y1p)
    c2 = cw * cw + ch * ch + EPS
    rho2 = ((x1t + x2t - x1p - x2p) ** 2 + (y1t + y2t - y1p - y2p) ** 2) / 4.0
    v = (4.0 / (np.pi ** 2)) * (_atan_nonneg(wt / (ht + EPS)) -
                                _atan_nonneg(wp / (hp + EPS))) ** 2
    alpha_t = v / (1.0 - iou2 + v + EPS)
    cl = 1.0 - (iou2 - rho2 / c2 - alpha_t * v)
    b_sum = jnp.sum(cl * pw)

    cnt = jnp.sum(pw)

    rows = jnp.concatenate([
        jnp.full((1, 128), s_sum, jnp.float32),
        jnp.full((1, 128), c_sum, jnp.float32),
        jnp.full((1, 128), b_sum, jnp.float32),
        jnp.full((1, 128), cnt, jnp.float32),
    ], axis=0)

    @pl.when(j == 0)
    def _init():
        out_ref[0] = rows

    @pl.when(j != 0)
    def _acc():
        out_ref[0] = out_ref[0] + rows


@jax.jit
def kernel(y_true, bbox_true, conf_pred, logit_pred, bbox_pred, anchors):
    ancT = jnp.transpose(anchors, (1, 0)).reshape(4, NBLK, 1, AB)
    btT = jnp.transpose(bbox_true, (0, 2, 1))
    conf3 = conf_pred.reshape(BATCH, NBLK, 1, AB)
    bpT = jnp.transpose(bbox_pred, (0, 2, 1)).reshape(BATCH, 4, NBLK, 1, AB)

    out = pl.pallas_call(
        _loss_block,
        grid=(BATCH, NBLK),
        in_specs=[
            pl.BlockSpec((4, 1, 1, AB), lambda b, j: (0, j, 0, 0)),
            pl.BlockSpec((1, MAX_TRUE, 4), lambda b, j: (b, 0, 0)),
            pl.BlockSpec((1, 4, MAX_TRUE), lambda b, j: (b, 0, 0)),
            pl.BlockSpec((1, MAX_TRUE, NUM_CLASSES), lambda b, j: (b, 0, 0)),
            pl.BlockSpec((1, 1, 1, AB), lambda b, j: (b, j, 0, 0)),
            pl.BlockSpec((1, AB, NUM_CLASSES), lambda b, j: (b, j, 0)),
            pl.BlockSpec((1, 4, 1, 1, AB), lambda b, j: (b, 0, j, 0, 0)),
        ],
        out_specs=pl.BlockSpec((1, 4, 128), lambda b, j: (b, 0, 0)),
        out_shape=jax.ShapeDtypeStruct((BATCH, 4, 128), jnp.float32),
    )(ancT, bbox_true, btT, y_true, conf3, logit_pred, bpT)

    sums = out[:, :, 0]
    avg = jnp.sum(jnp.maximum(sums[:, 3], 1.0))
    losses = jnp.stack([jnp.sum(sums[:, 0]), jnp.sum(sums[:, 1]),
                        jnp.sum(sums[:, 2])]) / avg
    return jnp.where(jnp.isnan(losses) | jnp.isinf(losses), 0.0, losses)
